# baseline (speedup 1.0000x reference)
import jax
import jax.numpy as jnp
from jax import lax
from jax.experimental import pallas as pl
from jax.experimental.pallas import tpu as pltpu

_KD = 6


def _body(logits_ref, feat_ref, out_ref, conf_ref, mt_ref):
    n = logits_ref.shape[1]

    @pl.when(pl.program_id(1) == 0)
    def _build():
        L = logits_ref[0]
        m = jnp.max(L, axis=0)
        ii = lax.broadcasted_iota(jnp.int32, (n, n), 0)
        preds = jnp.min(jnp.where(L == m[None, :], ii, n), axis=0)
        s = jnp.sum(jnp.exp(L - m[None, :]), axis=0)
        conf_ref[0, 0] = 1.0 / s

        pp = lax.broadcasted_iota(jnp.int32, (n, n), 1)
        jj = lax.broadcasted_iota(jnp.int32, (n, n), 0)
        winner = jnp.max(jnp.where(preds[:, None] == pp, jj, -1), axis=0)
        Pt = (jj == winner[None, :]).astype(jnp.float32)

        inner = (Pt[:, :-2] + Pt[:, 1:-1] + Pt[:, 2:]) * (1.0 / 3.0)
        mt_ref[...] = jnp.concatenate(
            [Pt[:, :1], inner, Pt[:, -1:]], axis=1).astype(jnp.bfloat16)

    out_ref[0] = lax.dot_general(
        feat_ref[0].astype(jnp.bfloat16), mt_ref[...],
        dimension_numbers=(((0,), (0,)), ((), ())),
        preferred_element_type=jnp.float32,
    )


def kernel(features, position_logits):
    b, n, d = features.shape
    dk = d // _KD
    recon_t, conf3 = pl.pallas_call(
        _body,
        grid=(b, _KD),
        in_specs=[
            pl.BlockSpec((1, n, n), lambda i, k: (i, 0, 0)),
            pl.BlockSpec((1, n, dk), lambda i, k: (i, 0, k)),
        ],
        out_specs=[
            pl.BlockSpec((1, dk, n), lambda i, k: (i, k, 0)),
            pl.BlockSpec((1, 1, n), lambda i, k: (i, 0, 0)),
        ],
        out_shape=[
            jax.ShapeDtypeStruct((b, d, n), jnp.float32),
            jax.ShapeDtypeStruct((b, 1, n), jnp.float32),
        ],
        scratch_shapes=[pltpu.VMEM((n, n), jnp.bfloat16)],
    )(position_logits, features)
    g = int(round(n ** 0.5))
    return (recon_t.reshape(b, d, g, g), conf3.reshape(b, n))


# speedup vs baseline: 1.9451x; 1.9451x over previous
import jax
import jax.numpy as jnp
from jax import lax
from jax.experimental import pallas as pl


def _body(logits_ref, feat_ref, out_ref, conf_ref):
    n = logits_ref.shape[1]
    L = logits_ref[0]
    m = jnp.max(L, axis=0)
    ii = lax.broadcasted_iota(jnp.int32, (n, n), 0)
    preds = jnp.min(jnp.where(L == m[None, :], ii, n), axis=0)
    s = jnp.sum(jnp.exp(L - m[None, :]), axis=0)
    conf_ref[0, 0] = 1.0 / s

    pp = lax.broadcasted_iota(jnp.int32, (n, n), 1)
    jj = lax.broadcasted_iota(jnp.int32, (n, n), 0)
    hit = preds[:, None] == pp
    winner = jnp.max(jnp.where(hit, jj, -1), axis=0)
    jj16 = lax.broadcasted_iota(jnp.int16, (n, n), 0)
    one = jnp.bfloat16(1.0)
    zero = jnp.bfloat16(0.0)
    Pt = jnp.where(jj16 == winner[None, :].astype(jnp.int16), one, zero)

    inner = (Pt[:, :-2] + Pt[:, 1:-1] + Pt[:, 2:]) * jnp.bfloat16(1.0 / 3.0)
    Mt = jnp.concatenate([Pt[:, :1], inner, Pt[:, -1:]], axis=1)

    out_ref[0] = lax.dot_general(
        feat_ref[0].astype(jnp.bfloat16), Mt,
        dimension_numbers=(((0,), (0,)), ((), ())),
        preferred_element_type=jnp.float32,
    )


def kernel(features, position_logits):
    b, n, d = features.shape
    recon_t, conf3 = pl.pallas_call(
        _body,
        grid=(b,),
        in_specs=[
            pl.BlockSpec((1, n, n), lambda i: (i, 0, 0)),
            pl.BlockSpec((1, n, d), lambda i: (i, 0, 0)),
        ],
        out_specs=[
            pl.BlockSpec((1, d, n), lambda i: (i, 0, 0)),
            pl.BlockSpec((1, 1, n), lambda i: (i, 0, 0)),
        ],
        out_shape=[
            jax.ShapeDtypeStruct((b, d, n), jnp.float32),
            jax.ShapeDtypeStruct((b, 1, n), jnp.float32),
        ],
    )(position_logits, features)
    g = int(round(n ** 0.5))
    return (recon_t.reshape(b, d, g, g), conf3.reshape(b, n))
